# baseline (speedup 1.0000x reference)
import functools

import jax
import jax.numpy as jnp
from jax import lax
from jax.experimental import pallas as pl
from jax.experimental.pallas import tpu as pltpu
from jax.experimental.pallas import tpu_sc as plsc

_K = 32
_NF = 100
_TW = _K + 1


def _sc_gather_t(ut_t, it_t, u_idx, i_idx):
    info = plsc.get_sparse_core_info()
    nc, ns = info.num_cores, info.num_subcores
    nw = nc * ns
    b = u_idx.shape[0]
    bpw = b // nw
    tw = ut_t.shape[0]
    mesh = plsc.VectorSubcoreMesh(core_axis_name="c", subcore_axis_name="s")
    ring = 8

    @functools.partial(
        pl.kernel,
        mesh=mesh,
        out_type=[jax.ShapeDtypeStruct((tw, b), jnp.float32),
                  jax.ShapeDtypeStruct((tw, b), jnp.float32)],
        scratch_types=[
            pltpu.VMEM((bpw + 16,), jnp.int32),
            pltpu.VMEM((bpw + 16,), jnp.int32),
            pltpu.VMEM((tw, bpw), jnp.float32),
            pltpu.VMEM((tw, bpw), jnp.float32),
        ]
        + [pltpu.VMEM((tw, 128), jnp.float32) for _ in range(2 * ring)]
        + [pltpu.SemaphoreType.DMA for _ in range(2 * ring)],
        compiler_params=pltpu.CompilerParams(needs_layout_passes=False),
    )
    def gather_kernel(ut_hbm, it_hbm, ui_hbm, ii_hbm, uo_hbm, io_hbm,
                      uidx_v, iidx_v, ucols_v, icols_v, *slots_sems):
        slots = slots_sems[:2 * ring]
        sems = slots_sems[2 * ring:]
        wid = lax.axis_index("s") * nc + lax.axis_index("c")
        base = wid * bpw
        pltpu.sync_copy(ui_hbm.at[pl.ds(base, bpw)], uidx_v.at[pl.ds(0, bpw)])
        pltpu.sync_copy(ii_hbm.at[pl.ds(base, bpw)], iidx_v.at[pl.ds(0, bpw)])

        rows0 = lax.broadcasted_iota(jnp.int32, (16,), 0)
        rows1 = rows0 + 16
        row32 = rows0 * 0 + _K
        lane0 = rows0 == 0

        def issue(t_hbm, v, sl):
            start = pl.multiple_of((v >> 7) << 7, 128)
            pltpu.async_copy(t_hbm.at[:, pl.ds(start, 128)], slots[sl],
                             sems[sl])

        uvec0 = uidx_v[pl.ds(0, 16)]
        ivec0 = iidx_v[pl.ds(0, 16)]
        for k in range(ring):
            issue(ut_hbm, uvec0[k], k)
            issue(it_hbm, ivec0[k], ring + k)

        def extract(t_hbm, cols_v, vec, sl, h, k):
            j = h * ring + k
            v = vec[k]
            pltpu.make_async_copy(t_hbm.at[:, pl.ds(0, 128)], slots[sl],
                                  sems[sl]).wait()
            offv = rows0 * 0 + (v & 127)
            c0 = plsc.load_gather(slots[sl], [rows0, offv])
            c1 = plsc.load_gather(slots[sl], [rows1, offv])
            c2 = plsc.load_gather(slots[sl], [row32, offv])
            jv = rows0 * 0 + j
            plsc.store_scatter(cols_v, [rows0, jv], c0)
            plsc.store_scatter(cols_v, [rows1, jv], c1)
            plsc.store_scatter(cols_v, [row32, jv], c2, mask=lane0)

            @pl.when(h < bpw // ring - 1)
            def _():
                issue(t_hbm, vec[k + ring], sl)

        def half_group(h, carry):
            uvec = uidx_v[pl.ds(h * ring, 16)]
            ivec = iidx_v[pl.ds(h * ring, 16)]
            for k in range(ring):
                extract(ut_hbm, ucols_v, uvec, k, h, k)
                extract(it_hbm, icols_v, ivec, ring + k, h, k)
            return carry

        lax.fori_loop(0, bpw // ring, half_group, 0)
        pltpu.sync_copy(ucols_v, uo_hbm.at[:, pl.ds(base, bpw)])
        pltpu.sync_copy(icols_v, io_hbm.at[:, pl.ds(base, bpw)])

    return gather_kernel(ut_t, it_t, u_idx, i_idx)


def _fm1_body(f_ref, vft_ref, vf_ref, wf_ref,
              v_ref, w_ref, sf_ref, qf_ref, wf1_ref):
    ft = f_ref[...]
    vft = vft_ref[...]
    p = lax.Precision.HIGHEST
    sf = jnp.dot(vft, ft, precision=p, preferred_element_type=jnp.float32)
    qf = jnp.dot(vft * vft, ft * ft, precision=p,
                 preferred_element_type=jnp.float32)
    w_feat = wf_ref[...] * ft
    w_ref[2:, :] = w_feat
    sf_ref[...] = sf
    qf_ref[...] = jnp.sum(qf, axis=0)[None, :]
    wf1_ref[...] = jnp.sum(w_feat, axis=0)[None, :]
    v_ref[2:, :, :] = vf_ref[...][:, :, None] * ft[:, None, :]


def _fm2_body(u_ref, i_ref, sf_ref, qf_ref, wf1_ref, vd_ref, wd_ref,
              v_ref, w_ref, s_ref):
    ut = u_ref[...]
    it = i_ref[...]
    uv = ut[:_K, :]
    iv = it[:_K, :]
    uw = ut[_K:_TW, :]
    iw = it[_K:_TW, :]
    s_sum = uv + iv + sf_ref[...]
    w_ref[...] = wd_ref[...]
    w_ref[0:1, :] = uw
    w_ref[1:2, :] = iw
    s_val = (uw[0, :] + iw[0, :] + wf1_ref[0, :]
             + 0.5 * (jnp.sum(s_sum * s_sum, axis=0)
                      - jnp.sum(uv * uv + iv * iv, axis=0)
                      - qf_ref[0, :]))
    s_ref[...] = s_val[None, :]
    v_ref[0:1, :, :] = uv[None, :, :]
    v_ref[1:2, :, :] = iv[None, :, :]


def kernel(u, i, feats, user_table, item_table, feat_table, w0):
    b = feats.shape[0]
    u_idx = u.reshape(b).astype(jnp.int32)
    i_idx = i.reshape(b).astype(jnp.int32)
    ut_t, it_t = _sc_gather_t(user_table.T, item_table.T, u_idx, i_idx)
    f_t = feats.T
    vf = feat_table[:, :_K]
    vft = vf.T
    wf = feat_table[:, _K:_TW]
    bb = 512
    nv = 2 + _NF
    vt1, wt1, sf, qf, wf1 = pl.pallas_call(
        _fm1_body,
        grid=(b // bb,),
        in_specs=[
            pl.BlockSpec((_NF, bb), lambda g: (0, g)),
            pl.BlockSpec((_K, _NF), lambda g: (0, 0)),
            pl.BlockSpec((_NF, _K), lambda g: (0, 0)),
            pl.BlockSpec((_NF, 1), lambda g: (0, 0)),
        ],
        out_specs=[
            pl.BlockSpec((nv, _K, bb), lambda g: (0, 0, g)),
            pl.BlockSpec((nv, bb), lambda g: (0, g)),
            pl.BlockSpec((_K, bb), lambda g: (0, g)),
            pl.BlockSpec((1, bb), lambda g: (0, g)),
            pl.BlockSpec((1, bb), lambda g: (0, g)),
        ],
        out_shape=[
            jax.ShapeDtypeStruct((nv, _K, b), jnp.float32),
            jax.ShapeDtypeStruct((nv, b), jnp.float32),
            jax.ShapeDtypeStruct((_K, b), jnp.float32),
            jax.ShapeDtypeStruct((1, b), jnp.float32),
            jax.ShapeDtypeStruct((1, b), jnp.float32),
        ],
        compiler_params=pltpu.CompilerParams(
            dimension_semantics=("parallel",)),
    )(f_t, vft, vf, wf)
    vt, wt, s2 = pl.pallas_call(
        _fm2_body,
        grid=(b // bb,),
        in_specs=[
            pl.BlockSpec((_TW, bb), lambda g: (0, g)),
            pl.BlockSpec((_TW, bb), lambda g: (0, g)),
            pl.BlockSpec((_K, bb), lambda g: (0, g)),
            pl.BlockSpec((1, bb), lambda g: (0, g)),
            pl.BlockSpec((1, bb), lambda g: (0, g)),
            pl.BlockSpec(memory_space=pl.ANY),
            pl.BlockSpec((nv, bb), lambda g: (0, g)),
        ],
        out_specs=[
            pl.BlockSpec((2, _K, bb), lambda g: (0, 0, g)),
            pl.BlockSpec((nv, bb), lambda g: (0, g)),
            pl.BlockSpec((1, bb), lambda g: (0, g)),
        ],
        out_shape=[
            jax.ShapeDtypeStruct((nv, _K, b), jnp.float32),
            jax.ShapeDtypeStruct((nv, b), jnp.float32),
            jax.ShapeDtypeStruct((1, b), jnp.float32),
        ],
        input_output_aliases={5: 0},
        compiler_params=pltpu.CompilerParams(
            dimension_semantics=("parallel",)),
    )(ut_t, it_t, sf, qf, wf1, vt1, wt1)
    s = s2.reshape(b) + w0
    w = wt.T
    v = vt.transpose(2, 0, 1)
    return (s, w, v)


# speedup vs baseline: 1.0009x; 1.0009x over previous
import functools

import jax
import jax.numpy as jnp
from jax import lax
from jax.experimental import pallas as pl
from jax.experimental.pallas import tpu as pltpu
from jax.experimental.pallas import tpu_sc as plsc

_K = 32
_NF = 100
_TW = _K + 1


def _sc_gather_t(ut_t, it_t, u_idx, i_idx):
    info = plsc.get_sparse_core_info()
    nc, ns = info.num_cores, info.num_subcores
    nw = nc * ns
    b = u_idx.shape[0]
    bpw = b // nw
    tw = ut_t.shape[0]
    mesh = plsc.VectorSubcoreMesh(core_axis_name="c", subcore_axis_name="s")
    ring = 8

    @functools.partial(
        pl.kernel,
        mesh=mesh,
        out_type=[jax.ShapeDtypeStruct((tw, b), jnp.float32),
                  jax.ShapeDtypeStruct((tw, b), jnp.float32)],
        scratch_types=[
            pltpu.VMEM((bpw + 16,), jnp.int32),
            pltpu.VMEM((bpw + 16,), jnp.int32),
            pltpu.VMEM((tw, bpw), jnp.float32),
            pltpu.VMEM((tw, bpw), jnp.float32),
        ]
        + [pltpu.VMEM((tw, 128), jnp.float32) for _ in range(2 * ring)]
        + [pltpu.SemaphoreType.DMA for _ in range(2 * ring)],
        compiler_params=pltpu.CompilerParams(needs_layout_passes=False),
    )
    def gather_kernel(ut_hbm, it_hbm, ui_hbm, ii_hbm, uo_hbm, io_hbm,
                      uidx_v, iidx_v, ucols_v, icols_v, *slots_sems):
        slots = slots_sems[:2 * ring]
        sems = slots_sems[2 * ring:]
        wid = lax.axis_index("s") * nc + lax.axis_index("c")
        base = wid * bpw
        pltpu.sync_copy(ui_hbm.at[pl.ds(base, bpw)], uidx_v.at[pl.ds(0, bpw)])
        pltpu.sync_copy(ii_hbm.at[pl.ds(base, bpw)], iidx_v.at[pl.ds(0, bpw)])

        rows0 = lax.broadcasted_iota(jnp.int32, (16,), 0)
        rows1 = rows0 + 16
        row32 = rows0 * 0 + _K
        lane0 = rows0 == 0

        def issue(t_hbm, v, sl):
            start = pl.multiple_of((v >> 7) << 7, 128)
            pltpu.async_copy(t_hbm.at[:, pl.ds(start, 128)], slots[sl],
                             sems[sl])

        uvec0 = uidx_v[pl.ds(0, 16)]
        ivec0 = iidx_v[pl.ds(0, 16)]
        for k in range(ring):
            issue(ut_hbm, uvec0[k], k)
            issue(it_hbm, ivec0[k], ring + k)

        def extract(t_hbm, cols_v, vec, sl, h, k):
            j = h * ring + k
            v = vec[k]
            pltpu.make_async_copy(t_hbm.at[:, pl.ds(0, 128)], slots[sl],
                                  sems[sl]).wait()
            offv = rows0 * 0 + (v & 127)
            c0 = plsc.load_gather(slots[sl], [rows0, offv])
            c1 = plsc.load_gather(slots[sl], [rows1, offv])
            c2 = plsc.load_gather(slots[sl], [row32, offv])
            jv = rows0 * 0 + j
            plsc.store_scatter(cols_v, [rows0, jv], c0)
            plsc.store_scatter(cols_v, [rows1, jv], c1)
            plsc.store_scatter(cols_v, [row32, jv], c2, mask=lane0)

            @pl.when(h < bpw // ring - 1)
            def _():
                issue(t_hbm, vec[k + ring], sl)

        def half_group(h, carry):
            uvec = uidx_v[pl.ds(h * ring, 16)]
            ivec = iidx_v[pl.ds(h * ring, 16)]
            for k in range(ring):
                extract(ut_hbm, ucols_v, uvec, k, h, k)
                extract(it_hbm, icols_v, ivec, ring + k, h, k)
            return carry

        lax.fori_loop(0, bpw // ring, half_group, 0)
        pltpu.sync_copy(ucols_v, uo_hbm.at[:, pl.ds(base, bpw)])
        pltpu.sync_copy(icols_v, io_hbm.at[:, pl.ds(base, bpw)])

    return gather_kernel(ut_t, it_t, u_idx, i_idx)


def _fm1_body(f_ref, vft_ref, vf_ref, wf_ref,
              v_ref, w_ref, sf_ref, qf_ref, wf1_ref):
    ft = f_ref[...]
    vft = vft_ref[...]
    p = lax.Precision.HIGHEST
    sf = jnp.dot(vft, ft, precision=p, preferred_element_type=jnp.float32)
    qf = jnp.dot(vft * vft, ft * ft, precision=p,
                 preferred_element_type=jnp.float32)
    w_feat = wf_ref[...] * ft
    w_ref[2:, :] = w_feat
    sf_ref[...] = sf
    qf_ref[...] = jnp.sum(qf, axis=0)[None, :]
    wf1_ref[...] = jnp.sum(w_feat, axis=0)[None, :]
    v_ref[2:, :, :] = vf_ref[...][:, :, None] * ft[:, None, :]


def _fm2_body(u_ref, i_ref, sf_ref, qf_ref, wf1_ref, vd_ref, wd_ref,
              v_ref, w_ref, s_ref):
    ut = u_ref[...]
    it = i_ref[...]
    uv = ut[:_K, :]
    iv = it[:_K, :]
    uw = ut[_K:_TW, :]
    iw = it[_K:_TW, :]
    s_sum = uv + iv + sf_ref[...]
    w_ref[...] = wd_ref[...]
    w_ref[0:1, :] = uw
    w_ref[1:2, :] = iw
    s_val = (uw[0, :] + iw[0, :] + wf1_ref[0, :]
             + 0.5 * (jnp.sum(s_sum * s_sum, axis=0)
                      - jnp.sum(uv * uv + iv * iv, axis=0)
                      - qf_ref[0, :]))
    s_ref[...] = s_val[None, :]
    v_ref[0:1, :, :] = uv[None, :, :]
    v_ref[1:2, :, :] = iv[None, :, :]


def kernel(u, i, feats, user_table, item_table, feat_table, w0):
    b = feats.shape[0]
    u_idx = u.reshape(b).astype(jnp.int32)
    i_idx = i.reshape(b).astype(jnp.int32)
    f_t = feats.T
    vf = feat_table[:, :_K]
    vft = vf.T
    wf = feat_table[:, _K:_TW]
    bb = 512
    nv = 2 + _NF
    vt1, wt1, sf, qf, wf1 = pl.pallas_call(
        _fm1_body,
        grid=(b // bb,),
        in_specs=[
            pl.BlockSpec((_NF, bb), lambda g: (0, g)),
            pl.BlockSpec((_K, _NF), lambda g: (0, 0)),
            pl.BlockSpec((_NF, _K), lambda g: (0, 0)),
            pl.BlockSpec((_NF, 1), lambda g: (0, 0)),
        ],
        out_specs=[
            pl.BlockSpec((nv, _K, bb), lambda g: (0, 0, g)),
            pl.BlockSpec((nv, bb), lambda g: (0, g)),
            pl.BlockSpec((_K, bb), lambda g: (0, g)),
            pl.BlockSpec((1, bb), lambda g: (0, g)),
            pl.BlockSpec((1, bb), lambda g: (0, g)),
        ],
        out_shape=[
            jax.ShapeDtypeStruct((nv, _K, b), jnp.float32),
            jax.ShapeDtypeStruct((nv, b), jnp.float32),
            jax.ShapeDtypeStruct((_K, b), jnp.float32),
            jax.ShapeDtypeStruct((1, b), jnp.float32),
            jax.ShapeDtypeStruct((1, b), jnp.float32),
        ],
        compiler_params=pltpu.CompilerParams(
            dimension_semantics=("parallel",)),
    )(f_t, vft, vf, wf)
    ut_t, it_t = _sc_gather_t(user_table.T, item_table.T, u_idx, i_idx)
    vt, wt, s2 = pl.pallas_call(
        _fm2_body,
        grid=(b // bb,),
        in_specs=[
            pl.BlockSpec((_TW, bb), lambda g: (0, g)),
            pl.BlockSpec((_TW, bb), lambda g: (0, g)),
            pl.BlockSpec((_K, bb), lambda g: (0, g)),
            pl.BlockSpec((1, bb), lambda g: (0, g)),
            pl.BlockSpec((1, bb), lambda g: (0, g)),
            pl.BlockSpec(memory_space=pl.ANY),
            pl.BlockSpec((nv, bb), lambda g: (0, g)),
        ],
        out_specs=[
            pl.BlockSpec((2, _K, bb), lambda g: (0, 0, g)),
            pl.BlockSpec((nv, bb), lambda g: (0, g)),
            pl.BlockSpec((1, bb), lambda g: (0, g)),
        ],
        out_shape=[
            jax.ShapeDtypeStruct((nv, _K, b), jnp.float32),
            jax.ShapeDtypeStruct((nv, b), jnp.float32),
            jax.ShapeDtypeStruct((1, b), jnp.float32),
        ],
        input_output_aliases={5: 0},
        compiler_params=pltpu.CompilerParams(
            dimension_semantics=("parallel",)),
    )(ut_t, it_t, sf, qf, wf1, vt1, wt1)
    s = s2.reshape(b) + w0
    w = wt.T
    v = vt.transpose(2, 0, 1)
    return (s, w, v)


# speedup vs baseline: 1.0385x; 1.0375x over previous
import functools

import jax
import jax.numpy as jnp
from jax import lax
from jax.experimental import pallas as pl
from jax.experimental.pallas import tpu as pltpu
from jax.experimental.pallas import tpu_sc as plsc

_K = 32
_NF = 100
_TW = _K + 1


def _sc_gather_t(ut_t, it_t, u_idx, i_idx):
    info = plsc.get_sparse_core_info()
    nc, ns = info.num_cores, info.num_subcores
    nw = nc * ns
    b = u_idx.shape[0]
    bpw = b // nw
    tw = ut_t.shape[0]
    mesh = plsc.VectorSubcoreMesh(core_axis_name="c", subcore_axis_name="s")
    ring = 8

    @functools.partial(
        pl.kernel,
        mesh=mesh,
        out_type=[jax.ShapeDtypeStruct((tw, b), jnp.float32),
                  jax.ShapeDtypeStruct((tw, b), jnp.float32)],
        scratch_types=[
            pltpu.VMEM((bpw + 16,), jnp.int32),
            pltpu.VMEM((bpw + 16,), jnp.int32),
            pltpu.VMEM((tw, bpw), jnp.float32),
            pltpu.VMEM((tw, bpw), jnp.float32),
        ]
        + [pltpu.VMEM((tw, 128), jnp.float32) for _ in range(2 * ring)]
        + [pltpu.SemaphoreType.DMA for _ in range(2 * ring)],
        compiler_params=pltpu.CompilerParams(needs_layout_passes=False),
    )
    def gather_kernel(ut_hbm, it_hbm, ui_hbm, ii_hbm, uo_hbm, io_hbm,
                      uidx_v, iidx_v, ucols_v, icols_v, *slots_sems):
        slots = slots_sems[:2 * ring]
        sems = slots_sems[2 * ring:]
        wid = lax.axis_index("s") * nc + lax.axis_index("c")
        base = wid * bpw
        pltpu.sync_copy(ui_hbm.at[pl.ds(base, bpw)], uidx_v.at[pl.ds(0, bpw)])
        pltpu.sync_copy(ii_hbm.at[pl.ds(base, bpw)], iidx_v.at[pl.ds(0, bpw)])

        rows0 = lax.broadcasted_iota(jnp.int32, (16,), 0)
        rows1 = rows0 + 16
        row32 = rows0 * 0 + _K
        lane0 = rows0 == 0

        def issue(t_hbm, v, sl):
            start = pl.multiple_of((v >> 7) << 7, 128)
            pltpu.async_copy(t_hbm.at[:, pl.ds(start, 128)], slots[sl],
                             sems[sl])

        uvec0 = uidx_v[pl.ds(0, 16)]
        ivec0 = iidx_v[pl.ds(0, 16)]
        for k in range(ring):
            issue(ut_hbm, uvec0[k], k)
            issue(it_hbm, ivec0[k], ring + k)

        def extract(t_hbm, cols_v, vec, sl, h, k):
            j = h * ring + k
            v = vec[k]
            pltpu.make_async_copy(t_hbm.at[:, pl.ds(0, 128)], slots[sl],
                                  sems[sl]).wait()
            offv = rows0 * 0 + (v & 127)
            c0 = plsc.load_gather(slots[sl], [rows0, offv])
            c1 = plsc.load_gather(slots[sl], [rows1, offv])
            c2 = plsc.load_gather(slots[sl], [row32, offv])
            jv = rows0 * 0 + j
            plsc.store_scatter(cols_v, [rows0, jv], c0)
            plsc.store_scatter(cols_v, [rows1, jv], c1)
            plsc.store_scatter(cols_v, [row32, jv], c2, mask=lane0)

            @pl.when(h < bpw // ring - 1)
            def _():
                issue(t_hbm, vec[k + ring], sl)

        def half_group(h, carry):
            uvec = uidx_v[pl.ds(h * ring, 16)]
            ivec = iidx_v[pl.ds(h * ring, 16)]
            for k in range(ring):
                extract(ut_hbm, ucols_v, uvec, k, h, k)
                extract(it_hbm, icols_v, ivec, ring + k, h, k)
            return carry

        lax.fori_loop(0, bpw // ring, half_group, 0)
        pltpu.sync_copy(ucols_v, uo_hbm.at[:, pl.ds(base, bpw)])
        pltpu.sync_copy(icols_v, io_hbm.at[:, pl.ds(base, bpw)])

    return gather_kernel(ut_t, it_t, u_idx, i_idx)


def _fm_body_t(u_ref, i_ref, f_ref, vft_ref, vf_ref, wf_ref,
               v_ref, w_ref, s_ref):
    ut = u_ref[...]
    it = i_ref[...]
    ft = f_ref[...]
    uv = ut[:_K, :]
    iv = it[:_K, :]
    uw = ut[_K:_TW, :]
    iw = it[_K:_TW, :]
    vft = vft_ref[...]
    p = lax.Precision.HIGHEST
    s_sum = uv + iv + jnp.dot(vft, ft, precision=p,
                              preferred_element_type=jnp.float32)
    s_sq = (uv * uv + iv * iv
            + jnp.dot(vft * vft, ft * ft, precision=p,
                      preferred_element_type=jnp.float32))
    w_feat = wf_ref[...] * ft
    w_ref[0:1, :] = uw
    w_ref[1:2, :] = iw
    w_ref[2:, :] = w_feat
    s_val = (uw[0, :] + iw[0, :] + jnp.sum(w_feat, axis=0)
             + 0.5 * jnp.sum(s_sum * s_sum - s_sq, axis=0))
    s_ref[...] = s_val[None, :]
    v_ref[0:1, :, :] = uv[None, :, :]
    v_ref[1:2, :, :] = iv[None, :, :]
    v_ref[2:, :, :] = vf_ref[...][:, :, None] * ft[:, None, :]


def kernel(u, i, feats, user_table, item_table, feat_table, w0):
    b = feats.shape[0]
    u_idx = u.reshape(b).astype(jnp.int32)
    i_idx = i.reshape(b).astype(jnp.int32)
    ut_t, it_t = _sc_gather_t(user_table.T, item_table.T, u_idx, i_idx)
    f_t = feats.T
    vf = feat_table[:, :_K]
    vft = vf.T
    wf = feat_table[:, _K:_TW]
    bb = 512
    vt, wt, s2 = pl.pallas_call(
        _fm_body_t,
        grid=(b // bb,),
        in_specs=[
            pl.BlockSpec((_TW, bb), lambda g: (0, g)),
            pl.BlockSpec((_TW, bb), lambda g: (0, g)),
            pl.BlockSpec((_NF, bb), lambda g: (0, g)),
            pl.BlockSpec((_K, _NF), lambda g: (0, 0)),
            pl.BlockSpec((_NF, _K), lambda g: (0, 0)),
            pl.BlockSpec((_NF, 1), lambda g: (0, 0)),
        ],
        out_specs=[
            pl.BlockSpec((2 + _NF, _K, bb), lambda g: (0, 0, g)),
            pl.BlockSpec((2 + _NF, bb), lambda g: (0, g)),
            pl.BlockSpec((1, bb), lambda g: (0, g)),
        ],
        out_shape=[
            jax.ShapeDtypeStruct((2 + _NF, _K, b), jnp.float32),
            jax.ShapeDtypeStruct((2 + _NF, b), jnp.float32),
            jax.ShapeDtypeStruct((1, b), jnp.float32),
        ],
        compiler_params=pltpu.CompilerParams(
            dimension_semantics=("parallel",)),
    )(ut_t, it_t, f_t, vft, vf, wf)
    s = s2.reshape(b) + w0
    w = wt.T
    v = vt.transpose(2, 0, 1)
    return (s, w, v)
